# baseline (speedup 1.0000x reference)
import functools

import jax
import jax.numpy as jnp
from jax import lax
from jax.experimental import pallas as pl
from jax.experimental.pallas import tpu as pltpu
from jax.experimental.pallas import tpu_sc as plsc

_D = 4096
_B = 8192
_NC = 2
_NS = 16
_NW = _NC * _NS
_BPW = _B // _NW
_C = 8
_NCHUNK = _BPW // _C

_mesh = plsc.VectorSubcoreMesh(
    core_axis_name="c", subcore_axis_name="s", num_cores=_NC, num_subcores=_NS
)


_NBUF = 3
_LEAD = 2


@functools.partial(
    pl.kernel,
    out_type=jax.ShapeDtypeStruct((_B, _D), jnp.float32),
    mesh=_mesh,
    scratch_types=[
        pltpu.VMEM((_BPW,), jnp.int32),
        pltpu.VMEM((_NBUF, _C, _D), jnp.float32),
        pltpu.SemaphoreType.DMA,
        pltpu.SemaphoreType.DMA,
        pltpu.SemaphoreType.DMA,
        pltpu.SemaphoreType.DMA,
        pltpu.SemaphoreType.DMA,
        pltpu.SemaphoreType.DMA,
    ],
    compiler_params=pltpu.CompilerParams(disable_bounds_checks=True),
)
def _sc_gather(
    table_hbm, idx_hbm, out_hbm, idx_v, buf_v, gs0, gs1, gs2, ws0, ws1, ws2
):
    gsems = (gs0, gs1, gs2)
    wsems = (ws0, ws1, ws2)
    wid = lax.axis_index("s") * _NC + lax.axis_index("c")
    base = wid * _BPW
    pltpu.sync_copy(idx_hbm.at[pl.ds(base, _BPW)], idx_v)

    def gather(g, b):
        return pltpu.make_async_copy(
            table_hbm.at[idx_v.at[pl.ds(g * _C, _C)]], buf_v.at[b], gsems[b]
        )

    def write(g, b):
        return pltpu.make_async_copy(
            buf_v.at[b], out_hbm.at[pl.ds(base + g * _C, _C)], wsems[b]
        )

    for g in range(_LEAD):
        gather(g, g % _NBUF).start()
    for g in range(_NCHUNK):
        gather(g, g % _NBUF).wait()
        write(g, g % _NBUF).start()
        h = g + _LEAD
        if h < _NCHUNK:
            if h >= _NBUF:
                write(h - _NBUF, h % _NBUF).wait()
            gather(h, h % _NBUF).start()
    for g in range(_NCHUNK - _NBUF, _NCHUNK):
        write(g, g % _NBUF).wait()


def kernel(x, pe):
    return _sc_gather(pe, x)


# speedup vs baseline: 1.0032x; 1.0032x over previous
import functools

import jax
import jax.numpy as jnp
from jax import lax
from jax.experimental import pallas as pl
from jax.experimental.pallas import tpu as pltpu
from jax.experimental.pallas import tpu_sc as plsc

_D = 4096
_B = 8192
_NC = 2
_NS = 16
_NW = _NC * _NS
_BPW = _B // _NW
_C = 8
_NCHUNK = _BPW // _C

_mesh = plsc.VectorSubcoreMesh(
    core_axis_name="c", subcore_axis_name="s", num_cores=_NC, num_subcores=_NS
)


_NBUF = 3
_LEAD = 2


@functools.partial(
    pl.kernel,
    out_type=jax.ShapeDtypeStruct((_B, _D), jnp.float32),
    mesh=_mesh,
    scratch_types=[
        pltpu.VMEM((_BPW,), jnp.int32),
        pltpu.VMEM((_NBUF, _C, _D), jnp.float32),
        pltpu.SemaphoreType.DMA,
        pltpu.SemaphoreType.DMA,
        pltpu.SemaphoreType.DMA,
        pltpu.SemaphoreType.DMA,
        pltpu.SemaphoreType.DMA,
        pltpu.SemaphoreType.DMA,
    ],
    compiler_params=pltpu.CompilerParams(
        disable_semaphore_checks=True, skip_device_barrier=True
    ),
)
def _sc_gather(
    table_hbm, idx_hbm, out_hbm, idx_v, buf_v, gs0, gs1, gs2, ws0, ws1, ws2
):
    gsems = (gs0, gs1, gs2)
    wsems = (ws0, ws1, ws2)
    wid = lax.axis_index("s") * _NC + lax.axis_index("c")
    base = wid * _BPW
    pltpu.sync_copy(idx_hbm.at[pl.ds(base, _BPW)], idx_v)

    def gather(g, b):
        return pltpu.make_async_copy(
            table_hbm.at[idx_v.at[pl.ds(g * _C, _C)]], buf_v.at[b], gsems[b]
        )

    def write(g, b):
        return pltpu.make_async_copy(
            buf_v.at[b], out_hbm.at[pl.ds(base + g * _C, _C)], wsems[b]
        )

    for g in range(_LEAD):
        gather(g, g % _NBUF).start()
    for g in range(_NCHUNK):
        gather(g, g % _NBUF).wait()
        write(g, g % _NBUF).start()
        h = g + _LEAD
        if h < _NCHUNK:
            if h >= _NBUF:
                write(h - _NBUF, h % _NBUF).wait()
            gather(h, h % _NBUF).start()
    for g in range(_NCHUNK - _NBUF, _NCHUNK):
        write(g, g % _NBUF).wait()


def kernel(x, pe):
    return _sc_gather(pe, x)


# speedup vs baseline: 1.0034x; 1.0002x over previous
import functools

import jax
import jax.numpy as jnp
from jax import lax
from jax.experimental import pallas as pl
from jax.experimental.pallas import tpu as pltpu
from jax.experimental.pallas import tpu_sc as plsc

_D = 4096
_B = 8192
_NC = 2
_NS = 16
_NW = _NC * _NS
_BPW = _B // _NW
_C = 8
_NCHUNK = _BPW // _C

_mesh = plsc.VectorSubcoreMesh(
    core_axis_name="c", subcore_axis_name="s", num_cores=_NC, num_subcores=_NS
)


_NBUF = 3
_LEAD = 2


@functools.partial(
    pl.kernel,
    out_type=jax.ShapeDtypeStruct((_B, _D), jnp.float32),
    mesh=_mesh,
    scratch_types=[
        pltpu.VMEM((_BPW,), jnp.int32),
        pltpu.VMEM((_NBUF, _C, _D), jnp.float32),
        pltpu.SemaphoreType.DMA,
        pltpu.SemaphoreType.DMA,
        pltpu.SemaphoreType.DMA,
        pltpu.SemaphoreType.DMA,
        pltpu.SemaphoreType.DMA,
        pltpu.SemaphoreType.DMA,
    ],
)
def _sc_gather(
    table_hbm, idx_hbm, out_hbm, idx_v, buf_v, gs0, gs1, gs2, ws0, ws1, ws2
):
    gsems = (gs0, gs1, gs2)
    wsems = (ws0, ws1, ws2)
    wid = lax.axis_index("s") * _NC + lax.axis_index("c")
    base = wid * _BPW
    pltpu.sync_copy(idx_hbm.at[pl.ds(base, _BPW)], idx_v)

    def gather(g, b):
        return pltpu.make_async_copy(
            table_hbm.at[idx_v.at[pl.ds(g * _C, _C)]], buf_v.at[b], gsems[b]
        )

    def write(g, b):
        return pltpu.make_async_copy(
            buf_v.at[b], out_hbm.at[pl.ds(base + g * _C, _C)], wsems[b]
        )

    for g in range(_LEAD):
        gather(g, g % _NBUF).start()
    for g in range(_NCHUNK):
        gather(g, g % _NBUF).wait()
        write(g, g % _NBUF).start()
        h = g + _LEAD
        if h < _NCHUNK:
            if h >= _NBUF:
                write(h - _NBUF, h % _NBUF).wait()
            gather(h, h % _NBUF).start()
    for g in range(_NCHUNK - _NBUF, _NCHUNK):
        write(g, g % _NBUF).wait()


def kernel(x, pe):
    return _sc_gather(pe, x)
